# baseline (speedup 1.0000x reference)
import functools

import jax
import jax.numpy as jnp
from jax import lax
from jax.experimental import pallas as pl
from jax.experimental.pallas import tpu as pltpu
from jax.experimental.pallas import tpu_sc as plsc

H, W = 256, 320
NPIX = H * W
FS = 320.0
SMOOTH_WEIGHT = 0.001
B, N = 4, 131072
NC, NS = 2, 16
BPC = B // NC
EV_PER_TILE = N // NS
CHUNK = 512
HCH = CHUNK // 128
NCHUNK = EV_PER_TILE // CHUNK
ACC_ROWS = BPC * 2 * 2 * NPIX
ZROWS = ACC_ROWS // NS
PIX_PER_TILE = NPIX // NS
RCHUNK = 1024
NRCHUNK = PIX_PER_TILE // RCHUNK


def _floorf(v):
    t = v.astype(jnp.int32).astype(jnp.float32)
    return jnp.where(t > v, t - 1.0, t)


def _issue_ev(base, bufs, sem, hbm):
    for j in range(4):
        pltpu.async_copy(hbm[j].at[pl.ds(base, CHUNK)], bufs[j], sem)


def _drain_ev(bufs, sem, hbm):
    dummy = hbm[0].at[pl.ds(0, CHUNK)]
    for r in bufs[0:4]:
        pltpu.make_async_copy(dummy, r, sem).wait()


def _fi_and_gather(b, bufs, gsem, hbm):
    evts, evy, evx, evp, fibY, fibX, fby, fbx, sidx, sval = bufs
    flow_hbm = hbm[4]

    for h in range(HCH):
        def fi_body(g, _, h=h):
            lanes = g * 16
            yv = evy[pl.ds(h * 128 + lanes, 16)]
            xv = evx[pl.ds(h * 128 + lanes, 16)]
            fiv = (yv * float(W) + xv).astype(jnp.int32)
            fibY[h, pl.ds(lanes, 16)] = fiv + (2 * b + 1) * NPIX
            fibX[h, pl.ds(lanes, 16)] = fiv + (2 * b) * NPIX
            return 0

        lax.fori_loop(0, 8, fi_body, 0)
    for h in range(HCH):
        pltpu.async_copy(flow_hbm.at[fibY.at[h]], fby.at[h], gsem)
        pltpu.async_copy(flow_hbm.at[fibX.at[h]], fbx.at[h], gsem)


def _drain_gather(bufs, gsem, dummy_hbm):
    dummy = dummy_hbm.at[0, 0]
    pltpu.make_async_copy(dummy, bufs[6], gsem).wait()
    pltpu.make_async_copy(dummy, bufs[7], gsem).wait()


def _corners(bl, bufs):
    evts, evy, evx, evp, fibY, fibX, fby, fbx, sidx, sval = bufs

    def grp_body(g, _, bl=bl, h=0):
        lanes = g * 16
        tsv = evts[pl.ds(h * 128 + lanes, 16)]
        yv = evy[pl.ds(h * 128 + lanes, 16)]
        xv = evx[pl.ds(h * 128 + lanes, 16)]
        pv = evp[pl.ds(h * 128 + lanes, 16)]
        fyF = fby[h, pl.ds(lanes, 16)] * FS
        fxF = fbx[h, pl.ds(lanes, 16)] * FS
        polN = pv.astype(jnp.int32) * NPIX
        tFy = tsv * fyF
        tFx = tsv * fxF
        for d in range(2):
            if d == 0:
                wy = yv + (fyF - tFy)
                wx = xv + (fxF - tFx)
                tsw = tsv
            else:
                wy = yv - tFy
                wx = xv - tFx
                tsw = 1.0 - tsv
            ty = (wy + 256.0).astype(jnp.int32).astype(jnp.float32) - 256.0
            tx = (wx + 256.0).astype(jnp.int32).astype(jnp.float32) - 256.0
            fracy = wy - ty
            fracx = wx - tx
            polbase = polN + (bl * 2 + d) * 2 * NPIX
            tyW = ty * float(W)
            my0 = (ty >= 0.0) & (ty < float(H))
            my1 = (ty >= -1.0) & (ty < float(H) - 1.0)
            mx0 = (tx >= 0.0) & (tx < float(W))
            mx1 = (tx >= -1.0) & (tx < float(W) - 1.0)
            wym = (jnp.where(my0, 1.0 - fracy, 0.0),
                   jnp.where(my1, fracy, 0.0))
            cyw = (jnp.where(my0, tyW, 0.0),
                   jnp.where(my1, tyW + float(W), 0.0))
            wxm = (jnp.where(mx0, 1.0 - fracx, 0.0),
                   jnp.where(mx1, fracx, 0.0))
            cxw = (jnp.where(mx0, tx, 0.0),
                   jnp.where(mx1, tx + 1.0, 0.0))
            for dy in range(2):
                for dx in range(2):
                    dc = d * 4 + dy * 2 + dx
                    w = wym[dy] * wxm[dx]
                    lin = (cyw[dy] + cxw[dx]).astype(jnp.int32) + polbase
                    sidx[dc, h, pl.ds(lanes, 16)] = lin
                    sval[dc, 0, h, pl.ds(lanes, 16)] = w
                    sval[dc, 1, h, pl.ds(lanes, 16)] = w * tsw
        return 0

    for h in range(HCH):
        lax.fori_loop(0, 8, functools.partial(grp_body, h=h), 0)


def _fire_scatters(sidx, sval, acc_cnt, acc_ts, sem):
    for dc in range(8):
        for h in range(HCH):
            pltpu.async_copy(
                sval.at[dc, 0, h], acc_cnt.at[sidx.at[dc, h]], sem, add=True)
            pltpu.async_copy(
                sval.at[dc, 1, h], acc_ts.at[sidx.at[dc, h]], sem, add=True)


def _ev_body(ts_hbm, y_hbm, x_hbm, p_hbm, flow_hbm, zeros_hbm, dummy_hbm,
             out_hbm,
             evtsA, evyA, evxA, evpA, fibYA, fibXA, fbyA, fbxA, sidxA, svalA,
             evtsB, evyB, evxB, evpB, fibYB, fibXB, fbyB, fbxB, sidxB, svalB,
             rcp0, rcn0, rtp0, rtn0, rcp1, rcn1, rtp1, rtn1,
             obuf, acc_cnt, acc_ts,
             ssem, semA, semB, evsemA, evsemB, gsemA, gsemB):
    c = lax.axis_index("c")
    s = lax.axis_index("s")
    wid = c * NS + s
    iota = lax.iota(jnp.int32, 16)
    hbm = (ts_hbm, y_hbm, x_hbm, p_hbm, flow_hbm)
    bufsA = (evtsA, evyA, evxA, evpA, fibYA, fibXA, fbyA, fbxA, sidxA, svalA)
    bufsB = (evtsB, evyB, evxB, evpB, fibYB, fibXB, fbyB, fbxB, sidxB, svalB)
    rsets = ((rcp0, rcn0, rtp0, rtn0), (rcp1, rcn1, rtp1, rtn1))

    pltpu.sync_copy(zeros_hbm, acc_cnt.at[pl.ds(s * ZROWS, ZROWS)])
    pltpu.sync_copy(zeros_hbm, acc_ts.at[pl.ds(s * ZROWS, ZROWS)])
    plsc.subcore_barrier()

    ev_limit = B * N - CHUNK
    for bl in range(BPC):
        b = c * BPC + bl
        ev_base = b * N + s * EV_PER_TILE
        _issue_ev(ev_base, bufsA, evsemA, hbm)

        def pair_body(k2, _, bl=bl, b=b, ev_base=ev_base):
            _drain_ev(bufsA, evsemA, hbm)
            _fi_and_gather(b, bufsA, gsemA, hbm)
            _issue_ev(ev_base + (2 * k2 + 1) * CHUNK, bufsB, evsemB, hbm)

            @pl.when(k2 > 0)
            def _():
                pltpu.make_async_copy(dummy_hbm, svalA, semA).wait()

            _drain_gather(bufsA, gsemA, dummy_hbm)
            _corners(bl, bufsA)
            _fire_scatters(sidxA, svalA, acc_cnt, acc_ts, semA)

            _drain_ev(bufsB, evsemB, hbm)
            _fi_and_gather(b, bufsB, gsemB, hbm)
            nxt = jnp.minimum(ev_base + (2 * k2 + 2) * CHUNK, ev_limit)
            _issue_ev(nxt, bufsA, evsemA, hbm)

            @pl.when(k2 > 0)
            def _():
                pltpu.make_async_copy(dummy_hbm, svalB, semB).wait()

            _drain_gather(bufsB, gsemB, dummy_hbm)
            _corners(bl, bufsB)
            _fire_scatters(sidxB, svalB, acc_cnt, acc_ts, semB)
            return 0

        lax.fori_loop(0, NCHUNK // 2, pair_body, 0)
        _drain_ev(bufsA, evsemA, hbm)
        pltpu.make_async_copy(dummy_hbm, svalA, semA).wait()
        pltpu.make_async_copy(dummy_hbm, svalB, semB).wait()

    plsc.subcore_barrier()

    pixbase = s * PIX_PER_TILE
    jobs = [(bl, d, kk)
            for bl in range(BPC) for d in range(2) for kk in range(NRCHUNK)]

    def _issue_rd(job, rset):
        bl, d, kk = job
        pr0 = (bl * 2 + d) * 2 * NPIX + pixbase + kk * RCHUNK
        pltpu.async_copy(acc_cnt.at[pl.ds(pr0, RCHUNK)], rset[0], ssem)
        pltpu.async_copy(acc_cnt.at[pl.ds(pr0 + NPIX, RCHUNK)], rset[1], ssem)
        pltpu.async_copy(acc_ts.at[pl.ds(pr0, RCHUNK)], rset[2], ssem)
        pltpu.async_copy(acc_ts.at[pl.ds(pr0 + NPIX, RCHUNK)], rset[3], ssem)

    def _drain_rd(rset):
        dummy = flow_hbm.at[pl.ds(0, RCHUNK)]
        for r in rset:
            pltpu.make_async_copy(dummy, r, ssem).wait()

    sums = {(bl, d): (jnp.zeros((16,), jnp.float32), jnp.zeros((16,), jnp.float32))
            for bl in range(BPC) for d in range(2)}
    _issue_rd(jobs[0], rsets[0])
    for i, job in enumerate(jobs):
        rset = rsets[i % 2]
        if i + 1 < len(jobs):
            _issue_rd(jobs[i + 1], rsets[(i + 1) % 2])
        _drain_rd(rset)
        rcp, rcn, rtp, rtn = rset

        def red_body(g, carry, rcp=rcp, rcn=rcn, rtp=rtp, rtn=rtn):
            sq, nz = carry
            lanes = g * 16
            wp = rcp[pl.ds(lanes, 16)]
            wn = rcn[pl.ds(lanes, 16)]
            tp = rtp[pl.ds(lanes, 16)]
            tn = rtn[pl.ds(lanes, 16)]
            ip = tp / (wp + 1e-9)
            inn = tn / (wn + 1e-9)
            sq = sq + ip * ip + inn * inn
            nz = nz + jnp.where((wp + wn) > 0.0, 1.0, 0.0)
            return (sq, nz)

        bl, d, kk = job
        sums[(bl, d)] = lax.fori_loop(0, RCHUNK // 16, red_body, sums[(bl, d)])

    for bl in range(BPC):
        for d in range(2):
            j = bl * 2 + d
            ssq, nnz = sums[(bl, d)]
            obuf[j, :] = ssq
            obuf[4 + j, :] = nnz
    pltpu.sync_copy(obuf, out_hbm.at[wid])


_STAGE_SET = [
    pltpu.VMEM((CHUNK,), jnp.float32),
    pltpu.VMEM((CHUNK,), jnp.float32),
    pltpu.VMEM((CHUNK,), jnp.float32),
    pltpu.VMEM((CHUNK,), jnp.float32),
    pltpu.VMEM((HCH, 128), jnp.int32),
    pltpu.VMEM((HCH, 128), jnp.int32),
    pltpu.VMEM((HCH, 128), jnp.float32),
    pltpu.VMEM((HCH, 128), jnp.float32),
    pltpu.VMEM((8, HCH, 128), jnp.int32),
    pltpu.VMEM((8, 2, HCH, 128), jnp.float32),
]
_RD_SET = [pltpu.VMEM((RCHUNK,), jnp.float32)] * 4
_EV_SCRATCH = _STAGE_SET + _STAGE_SET + _RD_SET + _RD_SET + [
    pltpu.VMEM((8, 16), jnp.float32),
    pltpu.VMEM_SHARED((ACC_ROWS,), jnp.float32),
    pltpu.VMEM_SHARED((ACC_ROWS,), jnp.float32),
    pltpu.SemaphoreType.DMA,
    pltpu.SemaphoreType.DMA,
    pltpu.SemaphoreType.DMA,
    pltpu.SemaphoreType.DMA,
    pltpu.SemaphoreType.DMA,
    pltpu.SemaphoreType.DMA,
    pltpu.SemaphoreType.DMA,
]
_EV_OUT = jax.ShapeDtypeStruct((NC * NS, 8, 16), jnp.float32)


@functools.cache
def _get_ev_kernel():
    mesh = plsc.VectorSubcoreMesh(
        core_axis_name="c", subcore_axis_name="s", num_cores=NC, num_subcores=NS)
    return pl.kernel(
        _ev_body,
        out_type=_EV_OUT,
        mesh=mesh,
        scratch_types=_EV_SCRATCH,
    )


def _smooth_body(fx_ref, fy_ref, out_ref):
    fx = fx_ref[...]
    fy = fy_ref[...]
    eps = 1e-6
    d_x = jnp.sqrt(((fx[:, :, :-1] - fx[:, :, 1:]) + (fy[:, :, :-1] - fy[:, :, 1:])) ** 2 + eps)
    d_y = jnp.sqrt(((fx[:, :-1, :] - fx[:, 1:, :]) + (fy[:, :-1, :] - fy[:, 1:, :])) ** 2 + eps)
    d_dr = jnp.sqrt(((fx[:, :-1, :-1] - fx[:, 1:, 1:]) + (fy[:, :-1, :-1] - fy[:, 1:, 1:])) ** 2 + eps)
    d_ur = jnp.sqrt(((fx[:, 1:, :-1] - fx[:, :-1, 1:]) + (fy[:, 1:, :-1] - fy[:, :-1, 1:])) ** 2 + eps)
    out_ref[0, 0] = (jnp.mean(d_x) + jnp.mean(d_y) + jnp.mean(d_dr) + jnp.mean(d_ur)) / 4.0


_smooth = pl.pallas_call(
    _smooth_body,
    out_shape=jax.ShapeDtypeStruct((1, 1), jnp.float32),
    out_specs=pl.BlockSpec(memory_space=pltpu.SMEM),
)


@jax.jit
def kernel(flow, event_list, pol_mask, event_mask):
    ts = event_list[:, :, 0].reshape(B * N)
    y = event_list[:, :, 1].reshape(B * N)
    x = event_list[:, :, 2].reshape(B * N)
    p = pol_mask[:, :, 0].reshape(B * N)
    flow_flat = flow.reshape(B * 2 * NPIX)
    zeros = jnp.zeros((ZROWS,), jnp.float32)
    dummy = jnp.zeros((8, 2, HCH, 128), jnp.float32)
    parts = _get_ev_kernel()(ts, y, x, p, flow_flat, zeros, dummy)
    parts = parts.reshape(NC, NS, 8, 16).sum(axis=(1, 3))
    ssq = parts[:, 0:4]
    nnz = parts[:, 4:8]
    ev_loss = jnp.sum(ssq / nnz)
    smooth = _smooth(flow[:, 0], flow[:, 1])[0, 0]
    return ev_loss + SMOOTH_WEIGHT * smooth


# speedup vs baseline: 1.0091x; 1.0091x over previous
import functools

import jax
import jax.numpy as jnp
from jax import lax
from jax.experimental import pallas as pl
from jax.experimental.pallas import tpu as pltpu
from jax.experimental.pallas import tpu_sc as plsc

H, W = 256, 320
NPIX = H * W
FS = 320.0
SMOOTH_WEIGHT = 0.001
B, N = 4, 131072
NC, NS = 2, 16
BPC = B // NC
EV_PER_TILE = N // NS
CHUNK = 256
HCH = CHUNK // 128
NCHUNK = EV_PER_TILE // CHUNK
ACC_ROWS = BPC * 2 * 2 * NPIX
ZROWS = ACC_ROWS // NS
PIX_PER_TILE = NPIX // NS
RCHUNK = 1024
NRCHUNK = PIX_PER_TILE // RCHUNK


def _floorf(v):
    t = v.astype(jnp.int32).astype(jnp.float32)
    return jnp.where(t > v, t - 1.0, t)


def _issue_ev(base, bufs, sem, hbm):
    for j in range(4):
        pltpu.async_copy(hbm[j].at[pl.ds(base, CHUNK)], bufs[j], sem)


def _drain_ev(bufs, sem, hbm):
    dummy = hbm[0].at[pl.ds(0, CHUNK)]
    for r in bufs[0:4]:
        pltpu.make_async_copy(dummy, r, sem).wait()


def _fi_and_gather(b, bufs, gsem, hbm):
    evts, evy, evx, evp, fibY, fibX, fby, fbx, sidx, sval = bufs
    flow_hbm = hbm[4]

    for h in range(HCH):
        def fi_body(g, _, h=h):
            lanes = g * 16
            yv = evy[pl.ds(h * 128 + lanes, 16)]
            xv = evx[pl.ds(h * 128 + lanes, 16)]
            fiv = (yv * float(W) + xv).astype(jnp.int32)
            fibY[h, pl.ds(lanes, 16)] = fiv + (2 * b + 1) * NPIX
            fibX[h, pl.ds(lanes, 16)] = fiv + (2 * b) * NPIX
            return 0

        lax.fori_loop(0, 8, fi_body, 0)
    for h in range(HCH):
        pltpu.async_copy(flow_hbm.at[fibY.at[h]], fby.at[h], gsem)
        pltpu.async_copy(flow_hbm.at[fibX.at[h]], fbx.at[h], gsem)


def _drain_gather(bufs, gsem, dummy_hbm):
    dummy = dummy_hbm.at[0, 0]
    pltpu.make_async_copy(dummy, bufs[6], gsem).wait()
    pltpu.make_async_copy(dummy, bufs[7], gsem).wait()


def _corners(bl, bufs):
    evts, evy, evx, evp, fibY, fibX, fby, fbx, sidx, sval = bufs

    def grp_body(g, _, bl=bl, h=0):
        lanes = g * 16
        tsv = evts[pl.ds(h * 128 + lanes, 16)]
        yv = evy[pl.ds(h * 128 + lanes, 16)]
        xv = evx[pl.ds(h * 128 + lanes, 16)]
        pv = evp[pl.ds(h * 128 + lanes, 16)]
        fyF = fby[h, pl.ds(lanes, 16)] * FS
        fxF = fbx[h, pl.ds(lanes, 16)] * FS
        polN = pv.astype(jnp.int32) * NPIX
        tFy = tsv * fyF
        tFx = tsv * fxF
        for d in range(2):
            if d == 0:
                wy = yv + (fyF - tFy)
                wx = xv + (fxF - tFx)
                tsw = tsv
            else:
                wy = yv - tFy
                wx = xv - tFx
                tsw = 1.0 - tsv
            ty = (wy + 256.0).astype(jnp.int32).astype(jnp.float32) - 256.0
            tx = (wx + 256.0).astype(jnp.int32).astype(jnp.float32) - 256.0
            fracy = wy - ty
            fracx = wx - tx
            polbase = polN + (bl * 2 + d) * 2 * NPIX
            tyW = ty * float(W)
            my0 = (ty >= 0.0) & (ty < float(H))
            my1 = (ty >= -1.0) & (ty < float(H) - 1.0)
            mx0 = (tx >= 0.0) & (tx < float(W))
            mx1 = (tx >= -1.0) & (tx < float(W) - 1.0)
            wym = (jnp.where(my0, 1.0 - fracy, 0.0),
                   jnp.where(my1, fracy, 0.0))
            cyw = (jnp.where(my0, tyW, 0.0),
                   jnp.where(my1, tyW + float(W), 0.0))
            wxm = (jnp.where(mx0, 1.0 - fracx, 0.0),
                   jnp.where(mx1, fracx, 0.0))
            cxw = (jnp.where(mx0, tx, 0.0),
                   jnp.where(mx1, tx + 1.0, 0.0))
            for dy in range(2):
                for dx in range(2):
                    dc = d * 4 + dy * 2 + dx
                    w = wym[dy] * wxm[dx]
                    lin = (cyw[dy] + cxw[dx]).astype(jnp.int32) + polbase
                    sidx[dc, h, pl.ds(lanes, 16)] = lin
                    sval[dc, 0, h, pl.ds(lanes, 16)] = w
                    sval[dc, 1, h, pl.ds(lanes, 16)] = w * tsw
        return 0

    for h in range(HCH):
        lax.fori_loop(0, 8, functools.partial(grp_body, h=h), 0)


def _fire_scatters(sidx, sval, acc_cnt, acc_ts, sem):
    for dc in range(8):
        for h in range(HCH):
            pltpu.async_copy(
                sval.at[dc, 0, h], acc_cnt.at[sidx.at[dc, h]], sem, add=True)
            pltpu.async_copy(
                sval.at[dc, 1, h], acc_ts.at[sidx.at[dc, h]], sem, add=True)


def _ev_body(ts_hbm, y_hbm, x_hbm, p_hbm, flow_hbm, zeros_hbm, dummy_hbm,
             out_hbm,
             evtsA, evyA, evxA, evpA, fibYA, fibXA, fbyA, fbxA, sidxA, svalA,
             evtsB, evyB, evxB, evpB, fibYB, fibXB, fbyB, fbxB, sidxB, svalB,
             rcp0, rcn0, rtp0, rtn0, rcp1, rcn1, rtp1, rtn1,
             obuf, acc_cnt, acc_ts,
             ssem, semA, semB, evsemA, evsemB, gsemA, gsemB):
    c = lax.axis_index("c")
    s = lax.axis_index("s")
    wid = c * NS + s
    iota = lax.iota(jnp.int32, 16)
    hbm = (ts_hbm, y_hbm, x_hbm, p_hbm, flow_hbm)
    bufsA = (evtsA, evyA, evxA, evpA, fibYA, fibXA, fbyA, fbxA, sidxA, svalA)
    bufsB = (evtsB, evyB, evxB, evpB, fibYB, fibXB, fbyB, fbxB, sidxB, svalB)
    rsets = ((rcp0, rcn0, rtp0, rtn0), (rcp1, rcn1, rtp1, rtn1))

    pltpu.sync_copy(zeros_hbm, acc_cnt.at[pl.ds(s * ZROWS, ZROWS)])
    pltpu.sync_copy(zeros_hbm, acc_ts.at[pl.ds(s * ZROWS, ZROWS)])
    plsc.subcore_barrier()

    ev_limit = B * N - CHUNK
    for bl in range(BPC):
        b = c * BPC + bl
        ev_base = b * N + s * EV_PER_TILE
        _issue_ev(ev_base, bufsA, evsemA, hbm)

        def pair_body(k2, _, bl=bl, b=b, ev_base=ev_base):
            _drain_ev(bufsA, evsemA, hbm)
            _fi_and_gather(b, bufsA, gsemA, hbm)
            _issue_ev(ev_base + (2 * k2 + 1) * CHUNK, bufsB, evsemB, hbm)

            @pl.when(k2 > 0)
            def _():
                pltpu.make_async_copy(dummy_hbm, svalA, semA).wait()

            _drain_gather(bufsA, gsemA, dummy_hbm)
            _corners(bl, bufsA)
            _fire_scatters(sidxA, svalA, acc_cnt, acc_ts, semA)

            _drain_ev(bufsB, evsemB, hbm)
            _fi_and_gather(b, bufsB, gsemB, hbm)
            nxt = jnp.minimum(ev_base + (2 * k2 + 2) * CHUNK, ev_limit)
            _issue_ev(nxt, bufsA, evsemA, hbm)

            @pl.when(k2 > 0)
            def _():
                pltpu.make_async_copy(dummy_hbm, svalB, semB).wait()

            _drain_gather(bufsB, gsemB, dummy_hbm)
            _corners(bl, bufsB)
            _fire_scatters(sidxB, svalB, acc_cnt, acc_ts, semB)
            return 0

        lax.fori_loop(0, NCHUNK // 2, pair_body, 0)
        _drain_ev(bufsA, evsemA, hbm)
        pltpu.make_async_copy(dummy_hbm, svalA, semA).wait()
        pltpu.make_async_copy(dummy_hbm, svalB, semB).wait()

    plsc.subcore_barrier()

    pixbase = s * PIX_PER_TILE
    jobs = [(bl, d, kk)
            for bl in range(BPC) for d in range(2) for kk in range(NRCHUNK)]

    def _issue_rd(job, rset):
        bl, d, kk = job
        pr0 = (bl * 2 + d) * 2 * NPIX + pixbase + kk * RCHUNK
        pltpu.async_copy(acc_cnt.at[pl.ds(pr0, RCHUNK)], rset[0], ssem)
        pltpu.async_copy(acc_cnt.at[pl.ds(pr0 + NPIX, RCHUNK)], rset[1], ssem)
        pltpu.async_copy(acc_ts.at[pl.ds(pr0, RCHUNK)], rset[2], ssem)
        pltpu.async_copy(acc_ts.at[pl.ds(pr0 + NPIX, RCHUNK)], rset[3], ssem)

    def _drain_rd(rset):
        dummy = flow_hbm.at[pl.ds(0, RCHUNK)]
        for r in rset:
            pltpu.make_async_copy(dummy, r, ssem).wait()

    sums = {(bl, d): (jnp.zeros((16,), jnp.float32), jnp.zeros((16,), jnp.float32))
            for bl in range(BPC) for d in range(2)}
    _issue_rd(jobs[0], rsets[0])
    for i, job in enumerate(jobs):
        rset = rsets[i % 2]
        if i + 1 < len(jobs):
            _issue_rd(jobs[i + 1], rsets[(i + 1) % 2])
        _drain_rd(rset)
        rcp, rcn, rtp, rtn = rset

        def red_body(g, carry, rcp=rcp, rcn=rcn, rtp=rtp, rtn=rtn):
            sq, nz = carry
            lanes = g * 16
            wp = rcp[pl.ds(lanes, 16)]
            wn = rcn[pl.ds(lanes, 16)]
            tp = rtp[pl.ds(lanes, 16)]
            tn = rtn[pl.ds(lanes, 16)]
            ip = tp / (wp + 1e-9)
            inn = tn / (wn + 1e-9)
            sq = sq + ip * ip + inn * inn
            nz = nz + jnp.where((wp + wn) > 0.0, 1.0, 0.0)
            return (sq, nz)

        bl, d, kk = job
        sums[(bl, d)] = lax.fori_loop(0, RCHUNK // 16, red_body, sums[(bl, d)])

    for bl in range(BPC):
        for d in range(2):
            j = bl * 2 + d
            ssq, nnz = sums[(bl, d)]
            obuf[j, :] = ssq
            obuf[4 + j, :] = nnz
    pltpu.sync_copy(obuf, out_hbm.at[wid])


_STAGE_SET = [
    pltpu.VMEM((CHUNK,), jnp.float32),
    pltpu.VMEM((CHUNK,), jnp.float32),
    pltpu.VMEM((CHUNK,), jnp.float32),
    pltpu.VMEM((CHUNK,), jnp.float32),
    pltpu.VMEM((HCH, 128), jnp.int32),
    pltpu.VMEM((HCH, 128), jnp.int32),
    pltpu.VMEM((HCH, 128), jnp.float32),
    pltpu.VMEM((HCH, 128), jnp.float32),
    pltpu.VMEM((8, HCH, 128), jnp.int32),
    pltpu.VMEM((8, 2, HCH, 128), jnp.float32),
]
_RD_SET = [pltpu.VMEM((RCHUNK,), jnp.float32)] * 4
_EV_SCRATCH = _STAGE_SET + _STAGE_SET + _RD_SET + _RD_SET + [
    pltpu.VMEM((8, 16), jnp.float32),
    pltpu.VMEM_SHARED((ACC_ROWS,), jnp.float32),
    pltpu.VMEM_SHARED((ACC_ROWS,), jnp.float32),
    pltpu.SemaphoreType.DMA,
    pltpu.SemaphoreType.DMA,
    pltpu.SemaphoreType.DMA,
    pltpu.SemaphoreType.DMA,
    pltpu.SemaphoreType.DMA,
    pltpu.SemaphoreType.DMA,
    pltpu.SemaphoreType.DMA,
]
_EV_OUT = jax.ShapeDtypeStruct((NC * NS, 8, 16), jnp.float32)


@functools.cache
def _get_ev_kernel():
    mesh = plsc.VectorSubcoreMesh(
        core_axis_name="c", subcore_axis_name="s", num_cores=NC, num_subcores=NS)
    return pl.kernel(
        _ev_body,
        out_type=_EV_OUT,
        mesh=mesh,
        scratch_types=_EV_SCRATCH,
    )


def _smooth_body(fx_ref, fy_ref, out_ref):
    fx = fx_ref[...]
    fy = fy_ref[...]
    eps = 1e-6
    d_x = jnp.sqrt(((fx[:, :, :-1] - fx[:, :, 1:]) + (fy[:, :, :-1] - fy[:, :, 1:])) ** 2 + eps)
    d_y = jnp.sqrt(((fx[:, :-1, :] - fx[:, 1:, :]) + (fy[:, :-1, :] - fy[:, 1:, :])) ** 2 + eps)
    d_dr = jnp.sqrt(((fx[:, :-1, :-1] - fx[:, 1:, 1:]) + (fy[:, :-1, :-1] - fy[:, 1:, 1:])) ** 2 + eps)
    d_ur = jnp.sqrt(((fx[:, 1:, :-1] - fx[:, :-1, 1:]) + (fy[:, 1:, :-1] - fy[:, :-1, 1:])) ** 2 + eps)
    out_ref[0, 0] = (jnp.mean(d_x) + jnp.mean(d_y) + jnp.mean(d_dr) + jnp.mean(d_ur)) / 4.0


_smooth = pl.pallas_call(
    _smooth_body,
    out_shape=jax.ShapeDtypeStruct((1, 1), jnp.float32),
    out_specs=pl.BlockSpec(memory_space=pltpu.SMEM),
)


@jax.jit
def kernel(flow, event_list, pol_mask, event_mask):
    ts = event_list[:, :, 0].reshape(B * N)
    y = event_list[:, :, 1].reshape(B * N)
    x = event_list[:, :, 2].reshape(B * N)
    p = pol_mask[:, :, 0].reshape(B * N)
    flow_flat = flow.reshape(B * 2 * NPIX)
    zeros = jnp.zeros((ZROWS,), jnp.float32)
    dummy = jnp.zeros((8, 2, HCH, 128), jnp.float32)
    parts = _get_ev_kernel()(ts, y, x, p, flow_flat, zeros, dummy)
    parts = parts.reshape(NC, NS, 8, 16).sum(axis=(1, 3))
    ssq = parts[:, 0:4]
    nnz = parts[:, 4:8]
    ev_loss = jnp.sum(ssq / nnz)
    smooth = _smooth(flow[:, 0], flow[:, 1])[0, 0]
    return ev_loss + SMOOTH_WEIGHT * smooth
